# Initial kernel scaffold; baseline (speedup 1.0000x reference)
import jax
import jax.numpy as jnp
from jax import lax
from jax.experimental import pallas as pl
from jax.experimental.pallas import tpu as pltpu
from jax.experimental.pallas import tpu_sc as plsc

N_NODES = 10000
N_EDGES = 320000
D_IN = 128
N_HEADS = 8
D_OUT = 16

NC = 2
NS = 16
NW = NC * NS
E_PER_W = N_EDGES // NW
CHUNK = 2000
N_CHUNKS = E_PER_W // CHUNK


def _tc_body(x_ref, w_ref, as_ref, at_ref, o_ref):
    xb = x_ref[...]
    w = w_ref[...]
    proj = lax.dot_general(
        xb, w, (((1,), (1,)), ((), ())),
        preferred_element_type=jnp.float32,
        precision=lax.Precision.HIGHEST,
    )
    jj = lax.broadcasted_iota(jnp.int32, (D_IN, N_HEADS), 0) >> 4
    hh = lax.broadcasted_iota(jnp.int32, (D_IN, N_HEADS), 1)
    sel = (jj == hh).astype(jnp.float32)
    s1 = lax.dot_general(
        proj * as_ref[...], sel, (((1,), (0,)), ((), ())),
        preferred_element_type=jnp.float32,
        precision=lax.Precision.HIGHEST,
    )
    s2 = lax.dot_general(
        proj * at_ref[...], sel, (((1,), (0,)), ((), ())),
        preferred_element_type=jnp.float32,
        precision=lax.Precision.HIGHEST,
    )
    u1 = lax.bitcast_convert_type(s1, jnp.uint32)
    u1 = (u1 + jnp.uint32(0x8000)) >> jnp.uint32(16)
    u2 = lax.bitcast_convert_type(s2, jnp.uint32)
    u2 = (u2 + jnp.uint32(0x8000)) & jnp.uint32(0xFFFF0000)
    o_ref[...] = lax.bitcast_convert_type(u1 | u2, jnp.int32)


_TC_ROWS = 1000
_tc_pack = pl.pallas_call(
    _tc_body,
    grid=(N_NODES // _TC_ROWS,),
    in_specs=[
        pl.BlockSpec((_TC_ROWS, D_IN), lambda i: (i, 0)),
        pl.BlockSpec((D_IN, D_IN), lambda i: (0, 0)),
        pl.BlockSpec((1, D_IN), lambda i: (0, 0)),
        pl.BlockSpec((1, D_IN), lambda i: (0, 0)),
    ],
    out_specs=pl.BlockSpec((_TC_ROWS, N_HEADS), lambda i: (i, 0)),
    out_shape=jax.ShapeDtypeStruct((N_NODES, N_HEADS), jnp.int32),
)


def _sc_body(tpack_hbm, ei_hbm, out_hbm, table_v, idx_s, idx_t, out_v):
    wid = lax.axis_index("s") * NC + lax.axis_index("c")
    pltpu.sync_copy(tpack_hbm, table_v)

    lanes = lax.iota(jnp.int32, 16)
    hi8 = lanes >> 3
    col = lanes & 7

    for c in range(N_CHUNKS):
        base = wid * E_PER_W + c * CHUNK
        pltpu.sync_copy(ei_hbm.at[0, pl.ds(base, CHUNK)], idx_s)
        pltpu.sync_copy(ei_hbm.at[1, pl.ds(base, CHUNK)], idx_t)

        def body(i, _):
            pair = 2 * i + hi8
            sv = plsc.load_gather(idx_s, [pair])
            tv = plsc.load_gather(idx_t, [pair])
            a = plsc.load_gather(table_v, [sv, col])
            b = plsc.load_gather(table_v, [tv, col])
            sa = plsc.bitcast(lax.shift_left(a, 16), jnp.float32)
            sb = plsc.bitcast(b & jnp.int32(-65536), jnp.float32)
            z = sa + sb
            sig = 1.0 / (1.0 + jnp.exp(-z))
            out_v[pl.ds(i * 16, 16)] = sig
            return 0

        lax.fori_loop(0, CHUNK // 2, body, 0)
        pltpu.sync_copy(out_v, out_hbm.at[pl.ds(base * N_HEADS, CHUNK * N_HEADS)])


_sc_edge = pl.kernel(
    _sc_body,
    out_type=jax.ShapeDtypeStruct((N_EDGES * N_HEADS,), jnp.float32),
    mesh=plsc.VectorSubcoreMesh(core_axis_name="c", subcore_axis_name="s"),
    scratch_types=[
        pltpu.VMEM((N_NODES, N_HEADS), jnp.int32),
        pltpu.VMEM((CHUNK,), jnp.int32),
        pltpu.VMEM((CHUNK,), jnp.int32),
        pltpu.VMEM((CHUNK * N_HEADS,), jnp.float32),
    ],
)


def kernel(x, edge_index, W, a_src, a_tgt):
    ei = edge_index.astype(jnp.int32)
    a_s = a_src.reshape(1, N_HEADS * D_OUT)
    a_t = a_tgt.reshape(1, N_HEADS * D_OUT)
    tpack = _tc_pack(x, W, a_s, a_t)
    out_flat = _sc_edge(tpack, ei)
    return out_flat.reshape(N_EDGES, N_HEADS)


# speedup vs baseline: 3.8279x; 3.8279x over previous
import jax
import jax.numpy as jnp
from jax import lax
from jax.experimental import pallas as pl
from jax.experimental.pallas import tpu as pltpu
from jax.experimental.pallas import tpu_sc as plsc

N_NODES = 10000
N_EDGES = 320000
D_IN = 128
N_HEADS = 8
D_OUT = 16

NC = 2
NS = 16
NW = NC * NS
E_PER_W = N_EDGES // NW
CHUNK = 2000
N_CHUNKS = E_PER_W // CHUNK


def _tc_body(x_ref, w_ref, as_ref, at_ref, o_ref):
    xb = x_ref[...]
    w = w_ref[...]
    proj = lax.dot_general(
        xb, w, (((1,), (1,)), ((), ())),
        preferred_element_type=jnp.float32,
        precision=lax.Precision.HIGHEST,
    )
    jj = lax.broadcasted_iota(jnp.int32, (D_IN, N_HEADS), 0) >> 4
    hh = lax.broadcasted_iota(jnp.int32, (D_IN, N_HEADS), 1)
    sel = (jj == hh).astype(jnp.float32)
    s1 = lax.dot_general(
        proj * as_ref[...], sel, (((1,), (0,)), ((), ())),
        preferred_element_type=jnp.float32,
        precision=lax.Precision.HIGHEST,
    )
    s2 = lax.dot_general(
        proj * at_ref[...], sel, (((1,), (0,)), ((), ())),
        preferred_element_type=jnp.float32,
        precision=lax.Precision.HIGHEST,
    )
    u1 = lax.bitcast_convert_type(s1, jnp.uint32)
    u1 = (u1 + jnp.uint32(0x8000)) >> jnp.uint32(16)
    u2 = lax.bitcast_convert_type(s2, jnp.uint32)
    u2 = (u2 + jnp.uint32(0x8000)) & jnp.uint32(0xFFFF0000)
    o_ref[...] = lax.bitcast_convert_type(u1 | u2, jnp.int32)


_TC_ROWS = 1000
_tc_pack = pl.pallas_call(
    _tc_body,
    grid=(N_NODES // _TC_ROWS,),
    in_specs=[
        pl.BlockSpec((_TC_ROWS, D_IN), lambda i: (i, 0)),
        pl.BlockSpec((D_IN, D_IN), lambda i: (0, 0)),
        pl.BlockSpec((1, D_IN), lambda i: (0, 0)),
        pl.BlockSpec((1, D_IN), lambda i: (0, 0)),
    ],
    out_specs=pl.BlockSpec((_TC_ROWS, N_HEADS), lambda i: (i, 0)),
    out_shape=jax.ShapeDtypeStruct((N_NODES, N_HEADS), jnp.int32),
)


def _sc_body(tpack_hbm, ei_hbm, out_hbm, table_v, idx_s, idx_t, out_v):
    wid = lax.axis_index("s") * NC + lax.axis_index("c")
    pltpu.sync_copy(tpack_hbm, table_v)

    lanes = lax.iota(jnp.int32, 16)
    hi8 = lanes >> 3
    col = lanes & 7

    for c in range(N_CHUNKS):
        base = wid * E_PER_W + c * CHUNK
        pltpu.sync_copy(ei_hbm.at[pl.ds(base, CHUNK)], idx_s)
        pltpu.sync_copy(ei_hbm.at[pl.ds(N_EDGES + base, CHUNK)], idx_t)

        def body(i, _):
            pair = 2 * i + hi8
            sv = plsc.load_gather(idx_s, [pair])
            tv = plsc.load_gather(idx_t, [pair])
            a = plsc.load_gather(table_v, [lax.shift_left(sv, 3) + col])
            b = plsc.load_gather(table_v, [lax.shift_left(tv, 3) + col])
            sa = plsc.bitcast(lax.shift_left(a, 16), jnp.float32)
            sb = plsc.bitcast(b & jnp.int32(-65536), jnp.float32)
            z = sa + sb
            sig = 1.0 / (1.0 + jnp.exp(-z))
            out_v[pl.ds(i * 16, 16)] = sig
            return 0

        lax.fori_loop(0, CHUNK // 2, body, 0)
        pltpu.sync_copy(out_v, out_hbm.at[pl.ds(base * N_HEADS, CHUNK * N_HEADS)])


_sc_edge = pl.kernel(
    _sc_body,
    out_type=jax.ShapeDtypeStruct((N_EDGES * N_HEADS,), jnp.float32),
    mesh=plsc.VectorSubcoreMesh(core_axis_name="c", subcore_axis_name="s"),
    compiler_params=pltpu.CompilerParams(needs_layout_passes=False),
    scratch_types=[
        pltpu.VMEM((N_NODES * N_HEADS,), jnp.int32),
        pltpu.VMEM((CHUNK,), jnp.int32),
        pltpu.VMEM((CHUNK,), jnp.int32),
        pltpu.VMEM((CHUNK * N_HEADS,), jnp.float32),
    ],
)


def kernel(x, edge_index, W, a_src, a_tgt):
    ei = edge_index.astype(jnp.int32).reshape(-1)
    a_s = a_src.reshape(1, N_HEADS * D_OUT)
    a_t = a_tgt.reshape(1, N_HEADS * D_OUT)
    tpack = _tc_pack(x, W, a_s, a_t)
    out_flat = _sc_edge(tpack.reshape(-1), ei)
    return out_flat.reshape(N_EDGES, N_HEADS)


# speedup vs baseline: 5.8252x; 1.5218x over previous
import jax
import jax.numpy as jnp
from jax import lax
from jax.experimental import pallas as pl
from jax.experimental.pallas import tpu as pltpu
from jax.experimental.pallas import tpu_sc as plsc

N_NODES = 10000
N_EDGES = 320000
D_IN = 128
N_HEADS = 8
D_OUT = 16

NC = 2
NS = 16
NW = NC * NS
E_PER_W = N_EDGES // NW
CHUNK = 2000
N_CHUNKS = E_PER_W // CHUNK


def _tc_body(x_ref, w_ref, as_ref, at_ref, o_ref):
    xb = x_ref[...]
    w = w_ref[...]
    proj = lax.dot_general(
        xb, w, (((1,), (1,)), ((), ())),
        preferred_element_type=jnp.float32,
        precision=lax.Precision.HIGHEST,
    )
    jj = lax.broadcasted_iota(jnp.int32, (D_IN, N_HEADS), 0) >> 4
    hh = lax.broadcasted_iota(jnp.int32, (D_IN, N_HEADS), 1)
    sel = (jj == hh).astype(jnp.float32)
    s1 = lax.dot_general(
        proj * as_ref[...], sel, (((1,), (0,)), ((), ())),
        preferred_element_type=jnp.float32,
        precision=lax.Precision.HIGHEST,
    )
    s2 = lax.dot_general(
        proj * at_ref[...], sel, (((1,), (0,)), ((), ())),
        preferred_element_type=jnp.float32,
        precision=lax.Precision.HIGHEST,
    )
    u1 = lax.bitcast_convert_type(s1, jnp.uint32)
    u1 = (u1 + jnp.uint32(0x8000)) >> jnp.uint32(16)
    u2 = lax.bitcast_convert_type(s2, jnp.uint32)
    u2 = (u2 + jnp.uint32(0x8000)) & jnp.uint32(0xFFFF0000)
    o_ref[...] = lax.bitcast_convert_type(u1 | u2, jnp.int32)


def _tc_split_body(ei_ref, s_ref, t_ref):
    s_ref[...] = ei_ref[0, :]
    t_ref[...] = ei_ref[1, :]


_tc_split = pl.pallas_call(
    _tc_split_body,
    out_shape=[
        jax.ShapeDtypeStruct((N_EDGES,), jnp.int32),
        jax.ShapeDtypeStruct((N_EDGES,), jnp.int32),
    ],
)


_TC_ROWS = 1000
_tc_pack = pl.pallas_call(
    _tc_body,
    grid=(N_NODES // _TC_ROWS,),
    in_specs=[
        pl.BlockSpec((_TC_ROWS, D_IN), lambda i: (i, 0)),
        pl.BlockSpec((D_IN, D_IN), lambda i: (0, 0)),
        pl.BlockSpec((1, D_IN), lambda i: (0, 0)),
        pl.BlockSpec((1, D_IN), lambda i: (0, 0)),
    ],
    out_specs=pl.BlockSpec((_TC_ROWS, N_HEADS), lambda i: (i, 0)),
    out_shape=jax.ShapeDtypeStruct((N_NODES, N_HEADS), jnp.int32),
)


def _sc_body(tpack_hbm, src_hbm, trg_hbm, out_hbm, table_v, idx_s, idx_t, out_v):
    wid = lax.axis_index("s") * NC + lax.axis_index("c")
    pltpu.sync_copy(tpack_hbm, table_v)

    lanes = lax.iota(jnp.int32, 16)
    hi8 = lanes >> 3
    col = lanes & 7

    for c in range(N_CHUNKS):
        base = wid * E_PER_W + c * CHUNK
        pltpu.sync_copy(src_hbm.at[pl.ds(base, CHUNK)], idx_s)
        pltpu.sync_copy(trg_hbm.at[pl.ds(base, CHUNK)], idx_t)

        @plsc.parallel_loop(0, CHUNK // 2, unroll=8)
        def _pair_loop(i):
            pair = 2 * i + hi8
            sv = plsc.load_gather(idx_s, [pair])
            tv = plsc.load_gather(idx_t, [pair])
            a = plsc.load_gather(table_v, [lax.shift_left(sv, 3) + col])
            b = plsc.load_gather(table_v, [lax.shift_left(tv, 3) + col])
            sa = plsc.bitcast(lax.shift_left(a, 16), jnp.float32)
            sb = plsc.bitcast(b & jnp.int32(-65536), jnp.float32)
            z = sa + sb
            sig = 1.0 / (1.0 + jnp.exp(-z))
            out_v[pl.ds(i * 16, 16)] = sig

        pltpu.sync_copy(out_v, out_hbm.at[pl.ds(base * N_HEADS, CHUNK * N_HEADS)])


_sc_edge = pl.kernel(
    _sc_body,
    out_type=jax.ShapeDtypeStruct((N_EDGES * N_HEADS,), jnp.float32),
    mesh=plsc.VectorSubcoreMesh(core_axis_name="c", subcore_axis_name="s"),
    compiler_params=pltpu.CompilerParams(needs_layout_passes=False),
    scratch_types=[
        pltpu.VMEM((N_NODES * N_HEADS,), jnp.int32),
        pltpu.VMEM((CHUNK,), jnp.int32),
        pltpu.VMEM((CHUNK,), jnp.int32),
        pltpu.VMEM((CHUNK * N_HEADS,), jnp.float32),
    ],
)


def kernel(x, edge_index, W, a_src, a_tgt):
    ei = edge_index.astype(jnp.int32)
    a_s = a_src.reshape(1, N_HEADS * D_OUT)
    a_t = a_tgt.reshape(1, N_HEADS * D_OUT)
    src, trg = _tc_split(ei)
    tpack = _tc_pack(x, W, a_s, a_t)
    out_flat = _sc_edge(tpack.reshape(-1), src, trg)
    return out_flat.reshape(N_EDGES, N_HEADS)


# speedup vs baseline: 10.2631x; 1.7618x over previous
import jax
import jax.numpy as jnp
from jax import lax
from jax.experimental import pallas as pl
from jax.experimental.pallas import tpu as pltpu
from jax.experimental.pallas import tpu_sc as plsc

N_NODES = 10000
N_EDGES = 320000
D_IN = 128
N_HEADS = 8
D_OUT = 16

NC = 2
NS = 16
NW = NC * NS
BLK = 128
N_BLOCKS = N_EDGES // BLK
BPW = N_BLOCKS // NW
N_EXTRA = N_BLOCKS - BPW * NW
CBLK = 13
N_CHUNKS = BPW // CBLK
CE = CBLK * BLK


def _tc_body(x_ref, w_ref, as_ref, at_ref, o_ref):
    xb = x_ref[...]
    w = w_ref[...]
    proj = lax.dot_general(
        xb, w, (((1,), (1,)), ((), ())),
        preferred_element_type=jnp.float32,
        precision=lax.Precision.HIGHEST,
    )
    jj = lax.broadcasted_iota(jnp.int32, (D_IN, N_HEADS), 0) >> 4
    hh = lax.broadcasted_iota(jnp.int32, (D_IN, N_HEADS), 1)
    sel = (jj == hh).astype(jnp.float32)
    s1 = lax.dot_general(
        proj * as_ref[...], sel, (((1,), (0,)), ((), ())),
        preferred_element_type=jnp.float32,
        precision=lax.Precision.HIGHEST,
    )
    s2 = lax.dot_general(
        proj * at_ref[...], sel, (((1,), (0,)), ((), ())),
        preferred_element_type=jnp.float32,
        precision=lax.Precision.HIGHEST,
    )
    u1 = lax.bitcast_convert_type(s1, jnp.uint32)
    u1 = (u1 + jnp.uint32(0x8000)) >> jnp.uint32(16)
    u2 = lax.bitcast_convert_type(s2, jnp.uint32)
    u2 = (u2 + jnp.uint32(0x8000)) & jnp.uint32(0xFFFF0000)
    o_ref[...] = lax.bitcast_convert_type(u1 | u2, jnp.int32)


def _tc_split_body(ei_ref, s_ref, t_ref):
    s_ref[...] = ei_ref[0, :]
    t_ref[...] = ei_ref[1, :]


_tc_split = pl.pallas_call(
    _tc_split_body,
    out_shape=[
        jax.ShapeDtypeStruct((N_EDGES,), jnp.int32),
        jax.ShapeDtypeStruct((N_EDGES,), jnp.int32),
    ],
)


_TC_ROWS = 1000
_tc_pack = pl.pallas_call(
    _tc_body,
    grid=(N_NODES // _TC_ROWS,),
    in_specs=[
        pl.BlockSpec((_TC_ROWS, D_IN), lambda i: (i, 0)),
        pl.BlockSpec((D_IN, D_IN), lambda i: (0, 0)),
        pl.BlockSpec((1, D_IN), lambda i: (0, 0)),
        pl.BlockSpec((1, D_IN), lambda i: (0, 0)),
    ],
    out_specs=pl.BlockSpec((_TC_ROWS, N_HEADS), lambda i: (i, 0)),
    out_shape=jax.ShapeDtypeStruct((N_NODES, N_HEADS), jnp.int32),
)


def _sc_body(tpack_hbm, src_hbm, trg_hbm, out_hbm, table_v, idx_s, idx_t, out_v):
    wid = lax.axis_index("s") * NC + lax.axis_index("c")
    pltpu.sync_copy(tpack_hbm, table_v)
    base_block = wid * BPW

    def do_chunk(gb0):
        pltpu.sync_copy(src_hbm.at[pl.ds(gb0 * BLK, CE)], idx_s)
        pltpu.sync_copy(trg_hbm.at[pl.ds(gb0 * BLK, CE)], idx_t)

        @plsc.parallel_loop(0, CBLK * 8, unroll=4)
        def _group_loop(g):
            sv = idx_s[pl.ds(g * 16, 16)]
            tv = idx_t[pl.ds(g * 16, 16)]
            sva = lax.shift_left(sv, 3)
            tva = lax.shift_left(tv, 3)
            off = lax.shift_left(lax.shift_right_logical(g, 3), 10) \
                + lax.shift_left(g & 7, 4)
            for h in range(N_HEADS):
                a = plsc.load_gather(table_v, [sva + h if h else sva])
                b = plsc.load_gather(table_v, [tva + h if h else tva])
                sa = plsc.bitcast(lax.shift_left(a, 16), jnp.float32)
                sb = plsc.bitcast(b & jnp.int32(-65536), jnp.float32)
                sig = 1.0 / (1.0 + jnp.exp(-(sa + sb)))
                out_v[pl.ds(off + h * BLK, 16)] = sig

        pltpu.sync_copy(out_v, out_hbm.at[pl.ds(gb0 * BLK * N_HEADS, CE * N_HEADS)])

    for c in range(N_CHUNKS):
        do_chunk(base_block + c * CBLK)
    do_chunk(N_BLOCKS - CBLK)


_sc_edge = pl.kernel(
    _sc_body,
    out_type=jax.ShapeDtypeStruct((N_EDGES * N_HEADS,), jnp.float32),
    mesh=plsc.VectorSubcoreMesh(core_axis_name="c", subcore_axis_name="s"),
    compiler_params=pltpu.CompilerParams(needs_layout_passes=False),
    scratch_types=[
        pltpu.VMEM((N_NODES * N_HEADS,), jnp.int32),
        pltpu.VMEM((CE,), jnp.int32),
        pltpu.VMEM((CE,), jnp.int32),
        pltpu.VMEM((CE * N_HEADS,), jnp.float32),
    ],
)


def kernel(x, edge_index, W, a_src, a_tgt):
    ei = edge_index.astype(jnp.int32)
    a_s = a_src.reshape(1, N_HEADS * D_OUT)
    a_t = a_tgt.reshape(1, N_HEADS * D_OUT)
    src, trg = _tc_split(ei)
    tpack = _tc_pack(x, W, a_s, a_t)
    out_flat = _sc_edge(tpack.reshape(-1), src, trg)
    return (out_flat.reshape(N_BLOCKS, N_HEADS, BLK)
            .swapaxes(1, 2)
            .reshape(N_EDGES, N_HEADS))
